# baseline (speedup 1.0000x reference)
import functools

import jax
import jax.numpy as jnp
from jax import lax
from jax.experimental import pallas as pl
from jax.experimental.pallas import tpu as pltpu
from jax.experimental.pallas import tpu_sc as plsc

D_MODEL = 128
BATCH = 1024
T = 200
B_TOTAL = BATCH * T
NC, NS = 2, 16
NW = NC * NS
B_PER_W = B_TOTAL // NW
CHUNK = 128
N_CHUNKS = B_PER_W // CHUNK
NBUF = 5
DG = 3
DS = 2


def _make_gather():
    mesh = plsc.VectorSubcoreMesh(core_axis_name="c", subcore_axis_name="s")

    @functools.partial(
        pl.kernel,
        mesh=mesh,
        out_type=jax.ShapeDtypeStruct((B_TOTAL, D_MODEL), jnp.float32),
        scratch_types=[
            pltpu.VMEM((B_PER_W,), jnp.int32),
        ]
        + [pltpu.VMEM((CHUNK, D_MODEL), jnp.float32)] * NBUF
        + [pltpu.SemaphoreType.DMA] * (2 * NBUF)
        + [pltpu.VMEM_SHARED((5000, D_MODEL), jnp.float32)],
    )
    def gather_kernel(idx_hbm, table_hbm, out_hbm, idx_v, *scratch):
        bufs = scratch[:NBUF]
        gsems = scratch[NBUF : 2 * NBUF]
        ssems = scratch[2 * NBUF : 3 * NBUF]
        table_sp = scratch[3 * NBUF]

        sid = lax.axis_index("s")
        wid = sid * NC + lax.axis_index("c")
        base = wid * B_PER_W

        pltpu.sync_copy(idx_hbm.at[pl.ds(base, B_PER_W)], idx_v)

        def fire_g(c, b, from_hbm=False):
            src = table_hbm if from_hbm else table_sp
            pltpu.async_copy(
                src.at[idx_v.at[pl.ds(c * CHUNK, CHUNK)]], bufs[b], gsems[b]
            )

        def wait_g(b):
            pltpu.make_async_copy(
                table_hbm.at[pl.ds(0, CHUNK)], bufs[b], gsems[b]
            ).wait()

        def fire_s(c, b):
            pltpu.async_copy(
                bufs[b], out_hbm.at[pl.ds(base + c * CHUNK, CHUNK)], ssems[b]
            )

        def wait_s(b):
            pltpu.make_async_copy(
                bufs[b], out_hbm.at[pl.ds(base, CHUNK)], ssems[b]
            ).wait()

        half = 2504

        @pl.when(sid == 0)
        def _():
            pltpu.sync_copy(table_hbm.at[pl.ds(0, half)], table_sp.at[pl.ds(0, half)])

        @pl.when(sid == 1)
        def _():
            pltpu.sync_copy(
                table_hbm.at[pl.ds(half, 5000 - half)],
                table_sp.at[pl.ds(half, 5000 - half)],
            )

        plsc.subcore_barrier()
        for c in range(DG):
            fire_g(c, c % NBUF)
        for c in range(DS):
            b = c % NBUF
            wait_g(b)
            fire_s(c, b)
            fire_g(c + DG, (c + DG) % NBUF)

        def steady(c, b):
            wait_g(b)
            fire_s(c, b)
            wait_s((b + NBUF - DS) % NBUF)
            fire_g(c + DG, (b + DG) % NBUF)

        n_mid = N_CHUNKS - DS - DG
        n_loop = n_mid // NBUF

        def ring(j, carry):
            for i in range(NBUF):
                c = DS + NBUF * j + i
                steady(c, (DS + i) % NBUF)
            return carry

        lax.fori_loop(0, n_loop, ring, 0)
        for c in range(DS + n_loop * NBUF, N_CHUNKS - DG):
            steady(c, c % NBUF)
        for c in range(N_CHUNKS - DG, N_CHUNKS):
            b = c % NBUF
            wait_g(b)
            fire_s(c, b)
            wait_s((b + NBUF - DS) % NBUF)
        for c in range(N_CHUNKS - DS, N_CHUNKS):
            wait_s(c % NBUF)

    return gather_kernel


_gather = _make_gather()


def kernel(delta_t, pe):
    idx = delta_t.reshape(-1)
    table = pe[0]
    out = _gather(idx, table)
    return out.reshape(BATCH, T, D_MODEL)


# speedup vs baseline: 1.0156x; 1.0156x over previous
import functools

import jax
import jax.numpy as jnp
from jax import lax
from jax.experimental import pallas as pl
from jax.experimental.pallas import tpu as pltpu
from jax.experimental.pallas import tpu_sc as plsc

D_MODEL = 128
BATCH = 1024
T = 200
B_TOTAL = BATCH * T
NC, NS = 2, 16
NW = NC * NS
B_PER_W = B_TOTAL // NW
CHUNK = 80
N_CHUNKS = B_PER_W // CHUNK
NBUF = 8
DG = 5
DS = 3


def _make_gather():
    mesh = plsc.VectorSubcoreMesh(core_axis_name="c", subcore_axis_name="s")

    @functools.partial(
        pl.kernel,
        mesh=mesh,
        out_type=jax.ShapeDtypeStruct((B_TOTAL, D_MODEL), jnp.float32),
        scratch_types=[
            pltpu.VMEM((B_PER_W,), jnp.int32),
        ]
        + [pltpu.VMEM((CHUNK, D_MODEL), jnp.float32)] * NBUF
        + [pltpu.SemaphoreType.DMA] * (2 * NBUF)
        + [pltpu.VMEM_SHARED((5000, D_MODEL), jnp.float32)],
    )
    def gather_kernel(idx_hbm, table_hbm, out_hbm, idx_v, *scratch):
        bufs = scratch[:NBUF]
        gsems = scratch[NBUF : 2 * NBUF]
        ssems = scratch[2 * NBUF : 3 * NBUF]
        table_sp = scratch[3 * NBUF]

        sid = lax.axis_index("s")
        wid = sid * NC + lax.axis_index("c")
        base = wid * B_PER_W

        pltpu.sync_copy(idx_hbm.at[pl.ds(base, B_PER_W)], idx_v)

        def fire_g(c, b, from_hbm=False):
            src = table_hbm if from_hbm else table_sp
            pltpu.async_copy(
                src.at[idx_v.at[pl.ds(c * CHUNK, CHUNK)]], bufs[b], gsems[b]
            )

        def wait_g(b):
            pltpu.make_async_copy(
                table_hbm.at[pl.ds(0, CHUNK)], bufs[b], gsems[b]
            ).wait()

        def fire_s(c, b):
            pltpu.async_copy(
                bufs[b], out_hbm.at[pl.ds(base + c * CHUNK, CHUNK)], ssems[b]
            )

        def wait_s(b):
            pltpu.make_async_copy(
                bufs[b], out_hbm.at[pl.ds(base, CHUNK)], ssems[b]
            ).wait()

        for c in range(DG):
            fire_g(c, c % NBUF, from_hbm=True)

        half = 2504

        @pl.when(sid == 0)
        def _():
            pltpu.sync_copy(table_hbm.at[pl.ds(0, half)], table_sp.at[pl.ds(0, half)])

        @pl.when(sid == 1)
        def _():
            pltpu.sync_copy(
                table_hbm.at[pl.ds(half, 5000 - half)],
                table_sp.at[pl.ds(half, 5000 - half)],
            )

        plsc.subcore_barrier()
        for c in range(DS):
            b = c % NBUF
            wait_g(b)
            fire_s(c, b)
            fire_g(c + DG, (c + DG) % NBUF)

        def steady(c, b):
            wait_g(b)
            fire_s(c, b)
            wait_s((b + NBUF - DS) % NBUF)
            fire_g(c + DG, (b + DG) % NBUF)

        n_mid = N_CHUNKS - DS - DG
        n_loop = n_mid // NBUF

        def ring(j, carry):
            for i in range(NBUF):
                c = DS + NBUF * j + i
                steady(c, (DS + i) % NBUF)
            return carry

        lax.fori_loop(0, n_loop, ring, 0)
        for c in range(DS + n_loop * NBUF, N_CHUNKS - DG):
            steady(c, c % NBUF)
        for c in range(N_CHUNKS - DG, N_CHUNKS):
            b = c % NBUF
            wait_g(b)
            fire_s(c, b)
            wait_s((b + NBUF - DS) % NBUF)
        for c in range(N_CHUNKS - DS, N_CHUNKS):
            wait_s(c % NBUF)

    return gather_kernel


_gather = _make_gather()


def kernel(delta_t, pe):
    idx = delta_t.reshape(-1)
    table = pe[0]
    out = _gather(idx, table)
    return out.reshape(BATCH, T, D_MODEL)
